# Initial kernel scaffold; baseline (speedup 1.0000x reference)
import jax
import jax.numpy as jnp
from jax.experimental import pallas as pl


def kernel(x, edges, input_layer, attention_vector):
    raise NotImplementedError("write your pallas kernel here")


# speedup vs baseline: 1.0003x; 1.0003x over previous
import jax
import jax.numpy as jnp
from jax.experimental import pallas as pl


def _mm_body(x_ref, w_ref, o_ref):
    o_ref[...] = jnp.dot(x_ref[...], w_ref[...],
                         preferred_element_type=jnp.float32)


def kernel(x, edges, input_layer, attention_vector):
    H, in_dim, L = input_layer.shape
    N = x.shape[0]
    E = edges.shape[0]
    HL = H * L

    Wc = jnp.transpose(input_layer, (1, 0, 2)).reshape(in_dim, HL)
    BN = 1000
    h2 = pl.pallas_call(
        _mm_body,
        grid=(N // BN,),
        in_specs=[
            pl.BlockSpec((BN, in_dim), lambda i: (i, 0)),
            pl.BlockSpec((in_dim, HL), lambda i: (0, 0)),
        ],
        out_specs=pl.BlockSpec((BN, HL), lambda i: (i, 0)),
        out_shape=jax.ShapeDtypeStruct((N, HL), jnp.float32),
    )(x, Wc)
    h = jnp.transpose(h2.reshape(N, H, L), (1, 0, 2))

    he = h[:, edges]
    h_concat = he.reshape(H, E, 2 * L)
    node_indices = edges[:, 0]
    aw = jnp.einsum('hf,hef->he', attention_vector, h_concat)
    aw = jax.nn.leaky_relu(aw, negative_slope=0.2)
    awT = aw.T
    seg_max = jax.ops.segment_max(awT, node_indices, num_segments=N)
    ex = jnp.exp(awT - seg_max[node_indices])
    seg_sum = jax.ops.segment_sum(ex, node_indices, num_segments=N)
    attn = (ex / seg_sum[node_indices]).T
    msg = attn[..., None] * he[:, :, 1]
    msg_e = jnp.moveaxis(msg, 1, 0)
    agg = jax.ops.segment_sum(msg_e, node_indices, num_segments=N)
    return agg.reshape(N, H * L)


# speedup vs baseline: 15.3556x; 15.3509x over previous
import functools

import jax
import jax.numpy as jnp
from jax import lax
from jax.experimental import pallas as pl
from jax.experimental.pallas import tpu as pltpu
from jax.experimental.pallas import tpu_sc as plsc

N = 10000
E = 160000
H = 4
L = 64
IND = 256

BN = 1000
EC = E // 16
CH = 400
CH5 = 80
NP = 20480
SL = NP // 16
NH = 5000
NHP = 5248
DUMP = 5120
YT = DUMP // 16
NYP = 10240


def _proj_body(x_ref, w_ref, h_ref, s_ref):
    xw = jnp.dot(x_ref[...], w_ref[...], preferred_element_type=jnp.float32)
    h_ref[...] = jnp.transpose(xw[:, :H * L].reshape(BN, 2, 2 * L), (1, 0, 2))
    s_ref[...] = xw[:, H * L:H * L + 8]


_mesh = plsc.VectorSubcoreMesh(core_axis_name="c", subcore_axis_name="s")


@functools.partial(
    pl.kernel,
    mesh=_mesh,
    compiler_params=pltpu.CompilerParams(needs_layout_passes=False),
    out_type=[
        jax.ShapeDtypeStruct((2, NYP, 2 * L), jnp.float32),
        jax.ShapeDtypeStruct((H * E,), jnp.float32),
    ],
    scratch_types=[
        pltpu.VMEM((N,), jnp.float32),
        pltpu.VMEM((N,), jnp.float32),
        pltpu.VMEM((NP,), jnp.float32),
        pltpu.VMEM((NP,), jnp.float32),
        pltpu.VMEM((CH,), jnp.int32),
        pltpu.VMEM((CH,), jnp.int32),
        pltpu.VMEM((CH,), jnp.float32),
        pltpu.VMEM((SL,), jnp.float32),
        pltpu.VMEM((CH5,), jnp.int32),
        pltpu.VMEM((1, CH5), jnp.int32),
        pltpu.VMEM((CH5, 2 * L), jnp.float32),
        pltpu.VMEM((2 * CH5,), jnp.float32),
        pltpu.VMEM_SHARED((8 * NP,), jnp.float32),
        pltpu.VMEM_SHARED((NHP, 2 * L), jnp.float32),
    ],
)
def _gat_edge(htab_hbm, s_hbm, e0_hbm, e1_hbm, y_hbm, aw_hbm,
              s1t, s2t, accm, accs, e0c, e1c, awc, tmp,
              gidx, sidx, rows, attnb, fold_sp, y_sp):
    c = lax.axis_index("c")
    t = lax.axis_index("s")
    ebase = t * EC

    neg = jnp.full((16,), -3.0e38, jnp.float32)
    zero = jnp.zeros((16,), jnp.float32)

    def init_body(i, _):
        accm[pl.ds(i * 16, 16)] = neg
        accs[pl.ds(i * 16, 16)] = zero
        return 0
    lax.fori_loop(0, NP // 16, init_body, 0)

    for hl in range(2):
        hh = 2 * c + hl
        pltpu.sync_copy(s_hbm.at[pl.ds(hh * N, N)], s1t)
        pltpu.sync_copy(s_hbm.at[pl.ds((4 + hh) * N, N)], s2t)

        def s1_chunk(k, _):
            pltpu.sync_copy(e0_hbm.at[pl.ds(ebase + k * CH, CH)], e0c)
            pltpu.sync_copy(e1_hbm.at[pl.ds(ebase + k * CH, CH)], e1c)

            def vec_body(v, _):
                i0 = e0c[pl.ds(v * 16, 16)]
                i1 = e1c[pl.ds(v * 16, 16)]
                g1 = plsc.load_gather(s1t, [i0])
                g2 = plsc.load_gather(s2t, [i1])
                aw = g1 + g2
                aw = jnp.where(aw > 0.0, aw, aw * jnp.float32(0.2))
                awc[pl.ds(v * 16, 16)] = aw
                midx = i0 * 2 + hl

                def wcond(it):
                    cur = plsc.load_gather(accm, [midx])
                    return jnp.any(aw > cur)

                def wbody(it):
                    cur = plsc.load_gather(accm, [midx])
                    m = aw > cur
                    plsc.store_scatter(accm, [midx], jnp.maximum(cur, aw),
                                       mask=m)
                    return it + 1
                lax.while_loop(wcond, wbody, 0)
                return 0
            lax.fori_loop(0, CH // 16, vec_body, 0)
            pltpu.sync_copy(
                awc, aw_hbm.at[pl.ds(hh * E + ebase + k * CH, CH)])
            return 0
        lax.fori_loop(0, EC // CH, s1_chunk, 0)

    def fold(acc, is_max):
        for step in (8, 4, 2, 1):
            plsc.subcore_barrier()

            @pl.when(jnp.logical_and(t >= step, t < 2 * step))
            def _():
                pltpu.sync_copy(acc, fold_sp.at[pl.ds((t - step) * NP, NP)])
            plsc.subcore_barrier()

            @pl.when(t < step)
            def _():
                def mrg(i, _):
                    pltpu.sync_copy(
                        fold_sp.at[pl.ds(t * NP + i * SL, SL)], tmp)

                    def mx(j, _):
                        a = acc[pl.ds(i * SL + j * 16, 16)]
                        b = tmp[pl.ds(j * 16, 16)]
                        acc[pl.ds(i * SL + j * 16, 16)] = (
                            jnp.maximum(a, b) if is_max else a + b)
                        return 0
                    lax.fori_loop(0, SL // 16, mx, 0)
                    return 0
                lax.fori_loop(0, 16, mrg, 0)
        plsc.subcore_barrier()

        @pl.when(t == 0)
        def _():
            pltpu.sync_copy(acc, fold_sp.at[pl.ds(0, NP)])
        plsc.subcore_barrier()
        pltpu.sync_copy(fold_sp.at[pl.ds(0, NP)], acc)

    fold(accm, True)

    for hl in range(2):
        hh = 2 * c + hl

        def s3_chunk(k, _):
            pltpu.sync_copy(e0_hbm.at[pl.ds(ebase + k * CH, CH)], e0c)
            pltpu.sync_copy(
                aw_hbm.at[pl.ds(hh * E + ebase + k * CH, CH)], awc)

            def vb(v, _):
                i0 = e0c[pl.ds(v * 16, 16)]
                aw = awc[pl.ds(v * 16, 16)]
                midx = i0 * 2 + hl
                m = plsc.load_gather(accm, [midx])
                plsc.addupdate_scatter(accs, [midx], jnp.exp(aw - m))
                return 0
            lax.fori_loop(0, CH // 16, vb, 0)
            return 0
        lax.fori_loop(0, EC // CH, s3_chunk, 0)

    fold(accs, False)

    for p in range(2):
        nbase = p * NH

        def zr(j, _):
            for qq in range(2 * L // 16):
                rows[j, pl.ds(qq * 16, 16)] = zero
            return 0
        lax.fori_loop(0, CH5, zr, 0)
        for i in range(YT // CH5):
            pltpu.sync_copy(rows, y_sp.at[pl.ds(t * YT + i * CH5, CH5)])
        plsc.subcore_barrier()

        def s5_chunk(q, _):
            base = ebase + q * CH5
            pltpu.sync_copy(e0_hbm.at[pl.ds(base, CH5)], sidx.at[0])
            pltpu.sync_copy(e1_hbm.at[pl.ds(base, CH5)], gidx)
            pltpu.sync_copy(aw_hbm.at[pl.ds(2 * c * E + base, CH5)],
                            awc.at[pl.ds(0, CH5)])
            pltpu.sync_copy(aw_hbm.at[pl.ds((2 * c + 1) * E + base, CH5)],
                            awc.at[pl.ds(CH5, CH5)])

            def att(v, _):
                i0 = sidx[0, pl.ds(v * 16, 16)]
                for hl in range(2):
                    aw = awc[pl.ds(hl * CH5 + v * 16, 16)]
                    midx = i0 * 2 + hl
                    mx = plsc.load_gather(accm, [midx])
                    sm = plsc.load_gather(accs, [midx])
                    attnb[pl.ds(hl * CH5 + v * 16, 16)] = (
                        jnp.exp(aw - mx) / sm)
                i0r = i0 - nbase
                inb = jnp.logical_and(i0r >= 0, i0r < NH)
                sidx[0, pl.ds(v * 16, 16)] = jnp.where(inb, i0r, DUMP)
                gidx[pl.ds(v * 16, 16)] = gidx[pl.ds(v * 16, 16)] + c * N
                return 0
            lax.fori_loop(0, CH5 // 16, att, 0)
            pltpu.sync_copy(htab_hbm.at[gidx], rows)
            for v in range(CH5 // 16):
                av0 = attnb[pl.ds(v * 16, 16)]
                av1 = attnb[pl.ds(CH5 + v * 16, 16)]
                for jj in range(16):
                    j = v * 16 + jj
                    a0 = av0[jj]
                    a1 = av1[jj]
                    for qq in range(4):
                        rows[j, pl.ds(qq * 16, 16)] = (
                            rows[j, pl.ds(qq * 16, 16)] * a0)
                    for qq in range(4, 8):
                        rows[j, pl.ds(qq * 16, 16)] = (
                            rows[j, pl.ds(qq * 16, 16)] * a1)
            pltpu.sync_copy(rows, y_sp.at[sidx.at[0]], add=True)
            return 0
        lax.fori_loop(0, EC // CH5, s5_chunk, 0)

        plsc.subcore_barrier()
        pltpu.sync_copy(y_sp.at[pl.ds(t * YT, YT)],
                        y_hbm.at[c].at[pl.ds(nbase + t * YT, YT)])
        plsc.subcore_barrier()


def kernel(x, edges, input_layer, attention_vector):
    a1 = attention_vector[:, :L]
    a2 = attention_vector[:, L:]
    Wc = jnp.transpose(input_layer, (1, 0, 2)).reshape(IND, H * L)
    v1 = jnp.einsum('hil,hl->ih', input_layer, a1)
    v2 = jnp.einsum('hil,hl->ih', input_layer, a2)
    Wf = jnp.concatenate([Wc, v1, v2], axis=1)

    h4, s = pl.pallas_call(
        _proj_body,
        grid=(N // BN,),
        in_specs=[
            pl.BlockSpec((BN, IND), lambda i: (i, 0)),
            pl.BlockSpec((IND, H * L + 8), lambda i: (0, 0)),
        ],
        out_specs=[
            pl.BlockSpec((2, BN, 2 * L), lambda i: (0, i, 0)),
            pl.BlockSpec((BN, 8), lambda i: (i, 0)),
        ],
        out_shape=[
            jax.ShapeDtypeStruct((2, N, 2 * L), jnp.float32),
            jax.ShapeDtypeStruct((N, 8), jnp.float32),
        ],
    )(x, Wf)

    htab = h4.reshape(2 * N, 2 * L)
    s_flat = jnp.transpose(s).reshape(8 * N)
    e0 = edges[:, 0].astype(jnp.int32)
    e1 = edges[:, 1].astype(jnp.int32)

    y4, _ = _gat_edge(htab, s_flat, e0, e1)
    return jnp.transpose(y4[:, :N, :], (1, 0, 2)).reshape(N, H * L)
